# baseline (speedup 1.0000x reference)
import functools

import jax
import jax.numpy as jnp
from jax import lax
from jax.experimental import pallas as pl
from jax.experimental.pallas import tpu as pltpu
from jax.experimental.pallas import tpu_sc as plsc

M_TILE = 8192
N_CHUNK = 1024
LANES = 128

SC_CORES = 2
SC_SUBCORES = 16
SC_WORKERS = SC_CORES * SC_SUBCORES
GATHER_CHUNK = 128


def _tc_body(n_emb, n_rows, z_ref, emb_ref, idx_ref, loss_ref, esq_ref, acc_ref):
    i = pl.program_id(0)
    n_chunks = n_emb // N_CHUNK
    numel = None

    @pl.when(i == 0)
    def _init():
        esq_ref[...] = 0.25 * jnp.sum(emb_ref[...] * emb_ref[...], axis=0)
        acc_ref[0] = 0.0

    z = z_ref[...]
    zsq = jnp.sum(z * z, axis=1)

    v_min = jnp.full((M_TILE, LANES), jnp.inf, dtype=jnp.float32)
    v_col = jnp.zeros((M_TILE, LANES), dtype=jnp.int32)
    for c in range(n_chunks):
        e = emb_ref[:, pl.ds(c * N_CHUNK, N_CHUNK)]
        s2 = lax.dot_general(z, e, (((1,), (0,)), ((), ())),
                             preferred_element_type=jnp.float32)
        esq_c = esq_ref[pl.ds(c * N_CHUNK, N_CHUNK)]
        d = (zsq[:, None] + s2) + esq_c[None, :]
        for k in range(N_CHUNK // LANES):
            dk = d[:, k * LANES:(k + 1) * LANES]
            col = c * (N_CHUNK // LANES) + k
            better = dk < v_min
            v_min = jnp.where(better, dk, v_min)
            v_col = jnp.where(better, col, v_col)

    lane = lax.broadcasted_iota(jnp.int32, (M_TILE, LANES), 1)
    j_full = v_col * LANES + lane
    m = jnp.min(v_min, axis=1)
    idx_row = jnp.min(jnp.where(v_min == m[:, None], j_full, jnp.int32(2 ** 30)),
                      axis=1)
    idx_ref[...] = idx_row
    acc_ref[0] += jnp.sum(m)

    numel = n_rows * z.shape[1]

    @pl.when(i == pl.num_programs(0) - 1)
    def _fin():
        loss_ref[0] = acc_ref[0] * (1.25 / numel)


def _tc_argmin_loss(z_flat, emb_scaled_t):
    n_rows, emb_dim = z_flat.shape
    n_emb = emb_scaled_t.shape[1]
    grid = (n_rows // M_TILE,)
    body = functools.partial(_tc_body, n_emb, n_rows)
    idx, loss = pl.pallas_call(
        body,
        grid=grid,
        in_specs=[
            pl.BlockSpec((M_TILE, emb_dim), lambda i: (i, 0)),
            pl.BlockSpec((emb_dim, n_emb), lambda i: (0, 0)),
        ],
        out_specs=[
            pl.BlockSpec((M_TILE,), lambda i: (i,)),
            pl.BlockSpec(memory_space=pltpu.SMEM),
        ],
        out_shape=[
            jax.ShapeDtypeStruct((n_rows,), jnp.int32),
            jax.ShapeDtypeStruct((1,), jnp.float32),
        ],
        scratch_shapes=[
            pltpu.VMEM((n_emb,), jnp.float32),
            pltpu.SMEM((1,), jnp.float32),
        ],
        compiler_params=pltpu.CompilerParams(
            dimension_semantics=("arbitrary",)),
    )(z_flat, emb_scaled_t)
    return idx, loss


def _sc_gather(emb_w, idx):
    n_rows = idx.shape[0]
    emb_dim = emb_w.shape[1]
    b_per_w = n_rows // SC_WORKERS
    n_chunks = b_per_w // GATHER_CHUNK
    mesh = plsc.VectorSubcoreMesh(core_axis_name="c", subcore_axis_name="s")

    @functools.partial(
        pl.kernel,
        mesh=mesh,
        out_type=jax.ShapeDtypeStruct((n_rows, emb_dim), jnp.float32),
        scratch_types=[
            pltpu.VMEM((GATHER_CHUNK,), jnp.int32),
            pltpu.VMEM((GATHER_CHUNK, emb_dim), jnp.float32),
            pltpu.SemaphoreType.DMA,
        ],
    )
    def gather(table_hbm, idx_hbm, out_hbm, idx_v, rows_v, sem):
        wid = lax.axis_index("s") * SC_CORES + lax.axis_index("c")
        base = wid * b_per_w
        for t in range(n_chunks):
            off = base + t * GATHER_CHUNK
            pltpu.sync_copy(idx_hbm.at[pl.ds(off, GATHER_CHUNK)], idx_v)
            pltpu.async_copy(table_hbm.at[idx_v], rows_v, sem).wait()
            pltpu.sync_copy(rows_v, out_hbm.at[pl.ds(off, GATHER_CHUNK)])

    return gather(emb_w, idx)


def kernel(z, emb_w):
    emb_dim = emb_w.shape[1]
    z_flat = z.reshape(-1, emb_dim)
    emb_scaled_t = (emb_w * (-2.0)).T
    idx, loss = _tc_argmin_loss(z_flat, emb_scaled_t)
    z_q = _sc_gather(emb_w, idx)
    return z_q.reshape(z.shape), loss[0], idx.reshape(z.shape[:-1])


# speedup vs baseline: 1.1714x; 1.1714x over previous
import functools

import jax
import jax.numpy as jnp
from jax import lax
from jax.experimental import pallas as pl
from jax.experimental.pallas import tpu as pltpu
from jax.experimental.pallas import tpu_sc as plsc

M_TILE = 4096
N_CHUNK = 1024
LANES = 128

SC_CORES = 2
SC_SUBCORES = 16
SC_WORKERS = SC_CORES * SC_SUBCORES
GATHER_CHUNK = 128


def _tc_body(n_emb, n_rows, z_ref, emb_ref, idx_ref, loss_ref, esq_ref, acc_ref):
    i = pl.program_id(0)
    n_chunks = n_emb // N_CHUNK
    numel = None

    @pl.when(i == 0)
    def _init():
        esq_ref[...] = 0.25 * jnp.sum(emb_ref[...] * emb_ref[...], axis=0)
        acc_ref[0] = 0.0

    z = z_ref[...]
    zsq = jnp.sum(z * z, axis=1)

    v_min = jnp.full((M_TILE, LANES), jnp.inf, dtype=jnp.float32)
    v_col = jnp.zeros((M_TILE, LANES), dtype=jnp.int32)
    for c in range(n_chunks):
        e = emb_ref[:, pl.ds(c * N_CHUNK, N_CHUNK)]
        s2 = lax.dot_general(z, e, (((1,), (0,)), ((), ())),
                             preferred_element_type=jnp.float32)
        esq_c = esq_ref[pl.ds(c * N_CHUNK, N_CHUNK)]
        d = (zsq[:, None] + s2) + esq_c[None, :]
        for k in range(N_CHUNK // LANES):
            dk = d[:, k * LANES:(k + 1) * LANES]
            col = c * (N_CHUNK // LANES) + k
            better = dk < v_min
            v_min = jnp.where(better, dk, v_min)
            v_col = jnp.where(better, col, v_col)

    lane = lax.broadcasted_iota(jnp.int32, (M_TILE, LANES), 1)
    j_full = v_col * LANES + lane
    m = jnp.min(v_min, axis=1)
    idx_row = jnp.min(jnp.where(v_min == m[:, None], j_full, jnp.int32(2 ** 30)),
                      axis=1)
    idx_ref[...] = idx_row
    acc_ref[0] += jnp.sum(m)

    numel = n_rows * z.shape[1]

    @pl.when(i == pl.num_programs(0) - 1)
    def _fin():
        loss_ref[0] = acc_ref[0] * (1.25 / numel)


def _tc_argmin_loss(z_flat, emb_scaled_t):
    n_rows, emb_dim = z_flat.shape
    n_emb = emb_scaled_t.shape[1]
    grid = (n_rows // M_TILE,)
    body = functools.partial(_tc_body, n_emb, n_rows)
    idx, loss = pl.pallas_call(
        body,
        grid=grid,
        in_specs=[
            pl.BlockSpec((M_TILE, emb_dim), lambda i: (i, 0)),
            pl.BlockSpec((emb_dim, n_emb), lambda i: (0, 0)),
        ],
        out_specs=[
            pl.BlockSpec((M_TILE,), lambda i: (i,)),
            pl.BlockSpec(memory_space=pltpu.SMEM),
        ],
        out_shape=[
            jax.ShapeDtypeStruct((n_rows,), jnp.int32),
            jax.ShapeDtypeStruct((1,), jnp.float32),
        ],
        scratch_shapes=[
            pltpu.VMEM((n_emb,), jnp.float32),
            pltpu.SMEM((1,), jnp.float32),
        ],
        compiler_params=pltpu.CompilerParams(
            dimension_semantics=("arbitrary",)),
    )(z_flat, emb_scaled_t)
    return idx, loss


def _sc_gather(emb_w, idx):
    n_rows = idx.shape[0]
    emb_dim = emb_w.shape[1]
    b_per_w = n_rows // SC_WORKERS
    n_chunks = b_per_w // GATHER_CHUNK
    mesh = plsc.VectorSubcoreMesh(core_axis_name="c", subcore_axis_name="s")

    @functools.partial(
        pl.kernel,
        mesh=mesh,
        out_type=jax.ShapeDtypeStruct((n_rows, emb_dim), jnp.float32),
        scratch_types=[
            pltpu.VMEM((b_per_w,), jnp.int32),
            pltpu.VMEM((GATHER_CHUNK, emb_dim), jnp.float32),
            pltpu.VMEM((GATHER_CHUNK, emb_dim), jnp.float32),
            pltpu.SemaphoreType.DMA,
            pltpu.SemaphoreType.DMA,
            pltpu.SemaphoreType.DMA,
            pltpu.SemaphoreType.DMA,
        ],
    )
    def gather(table_hbm, idx_hbm, out_hbm, idx_v, rb0, rb1, g0, g1, s0, s1):
        rb = (rb0, rb1)
        gsem = (g0, g1)
        ssem = (s0, s1)
        wid = lax.axis_index("s") * SC_CORES + lax.axis_index("c")
        base = wid * b_per_w
        pltpu.sync_copy(idx_hbm.at[pl.ds(base, b_per_w)], idx_v)

        def start_gather(t):
            return pltpu.async_copy(
                table_hbm.at[idx_v.at[pl.ds(t * GATHER_CHUNK, GATHER_CHUNK)]],
                rb[t % 2], gsem[t % 2])

        gathers = {0: start_gather(0)}
        stores = {}
        for t in range(n_chunks):
            if t + 1 < n_chunks:
                if t >= 1:
                    stores[t - 1].wait()
                gathers[t + 1] = start_gather(t + 1)
            gathers[t].wait()
            stores[t] = pltpu.async_copy(
                rb[t % 2],
                out_hbm.at[pl.ds(base + t * GATHER_CHUNK, GATHER_CHUNK)],
                ssem[t % 2])
        stores[n_chunks - 2].wait()
        stores[n_chunks - 1].wait()

    return gather(emb_w, idx)


def kernel(z, emb_w):
    emb_dim = emb_w.shape[1]
    z_flat = z.reshape(-1, emb_dim)
    emb_scaled_t = (emb_w * (-2.0)).T
    idx, loss = _tc_argmin_loss(z_flat, emb_scaled_t)
    z_q = _sc_gather(emb_w, idx)
    return z_q.reshape(z.shape), loss[0], idx.reshape(z.shape[:-1])


# speedup vs baseline: 1.1721x; 1.0006x over previous
import functools

import jax
import jax.numpy as jnp
from jax import lax
from jax.experimental import pallas as pl
from jax.experimental.pallas import tpu as pltpu
from jax.experimental.pallas import tpu_sc as plsc

M_TILE = 4096
N_CHUNK = 2048
LANES = 128

SC_CORES = 2
SC_SUBCORES = 16
SC_WORKERS = SC_CORES * SC_SUBCORES
GATHER_CHUNK = 128


def _tc_body(n_emb, n_rows, z_ref, emb_ref, idx_ref, loss_ref, esq_ref, acc_ref):
    i = pl.program_id(0)
    n_chunks = n_emb // N_CHUNK
    numel = None

    @pl.when(i == 0)
    def _init():
        esq_ref[...] = 0.25 * jnp.sum(emb_ref[...] * emb_ref[...], axis=0)
        acc_ref[0] = 0.0

    z = z_ref[...]
    zsq = jnp.sum(z * z, axis=1)

    v_min = jnp.full((M_TILE, LANES), jnp.inf, dtype=jnp.float32)
    v_col = jnp.zeros((M_TILE, LANES), dtype=jnp.int32)
    for c in range(n_chunks):
        e = emb_ref[:, pl.ds(c * N_CHUNK, N_CHUNK)]
        s2 = lax.dot_general(z, e, (((1,), (0,)), ((), ())),
                             preferred_element_type=jnp.float32)
        esq_c = esq_ref[pl.ds(c * N_CHUNK, N_CHUNK)]
        d = (zsq[:, None] + s2) + esq_c[None, :]
        for k in range(N_CHUNK // LANES):
            dk = d[:, k * LANES:(k + 1) * LANES]
            col = c * (N_CHUNK // LANES) + k
            better = dk < v_min
            v_min = jnp.where(better, dk, v_min)
            v_col = jnp.where(better, col, v_col)

    lane = lax.broadcasted_iota(jnp.int32, (M_TILE, LANES), 1)
    j_full = v_col * LANES + lane
    m = jnp.min(v_min, axis=1)
    idx_row = jnp.min(jnp.where(v_min == m[:, None], j_full, jnp.int32(2 ** 30)),
                      axis=1)
    idx_ref[...] = idx_row
    acc_ref[0] += jnp.sum(m)

    numel = n_rows * z.shape[1]

    @pl.when(i == pl.num_programs(0) - 1)
    def _fin():
        loss_ref[0] = acc_ref[0] * (1.25 / numel)


def _tc_argmin_loss(z_flat, emb_scaled_t):
    n_rows, emb_dim = z_flat.shape
    n_emb = emb_scaled_t.shape[1]
    grid = (n_rows // M_TILE,)
    body = functools.partial(_tc_body, n_emb, n_rows)
    idx, loss = pl.pallas_call(
        body,
        grid=grid,
        in_specs=[
            pl.BlockSpec((M_TILE, emb_dim), lambda i: (i, 0)),
            pl.BlockSpec((emb_dim, n_emb), lambda i: (0, 0)),
        ],
        out_specs=[
            pl.BlockSpec((M_TILE,), lambda i: (i,)),
            pl.BlockSpec(memory_space=pltpu.SMEM),
        ],
        out_shape=[
            jax.ShapeDtypeStruct((n_rows,), jnp.int32),
            jax.ShapeDtypeStruct((1,), jnp.float32),
        ],
        scratch_shapes=[
            pltpu.VMEM((n_emb,), jnp.float32),
            pltpu.SMEM((1,), jnp.float32),
        ],
        compiler_params=pltpu.CompilerParams(
            dimension_semantics=("arbitrary",)),
    )(z_flat, emb_scaled_t)
    return idx, loss


def _sc_gather(emb_w, idx):
    n_rows = idx.shape[0]
    emb_dim = emb_w.shape[1]
    b_per_w = n_rows // SC_WORKERS
    n_chunks = b_per_w // GATHER_CHUNK
    mesh = plsc.VectorSubcoreMesh(core_axis_name="c", subcore_axis_name="s")

    @functools.partial(
        pl.kernel,
        mesh=mesh,
        out_type=jax.ShapeDtypeStruct((n_rows, emb_dim), jnp.float32),
        scratch_types=[
            pltpu.VMEM((b_per_w,), jnp.int32),
            pltpu.VMEM((GATHER_CHUNK, emb_dim), jnp.float32),
            pltpu.VMEM((GATHER_CHUNK, emb_dim), jnp.float32),
            pltpu.SemaphoreType.DMA,
            pltpu.SemaphoreType.DMA,
            pltpu.SemaphoreType.DMA,
            pltpu.SemaphoreType.DMA,
        ],
    )
    def gather(table_hbm, idx_hbm, out_hbm, idx_v, rb0, rb1, g0, g1, s0, s1):
        rb = (rb0, rb1)
        gsem = (g0, g1)
        ssem = (s0, s1)
        wid = lax.axis_index("s") * SC_CORES + lax.axis_index("c")
        base = wid * b_per_w
        pltpu.sync_copy(idx_hbm.at[pl.ds(base, b_per_w)], idx_v)

        def start_gather(t):
            return pltpu.async_copy(
                table_hbm.at[idx_v.at[pl.ds(t * GATHER_CHUNK, GATHER_CHUNK)]],
                rb[t % 2], gsem[t % 2])

        gathers = {0: start_gather(0)}
        stores = {}
        for t in range(n_chunks):
            if t + 1 < n_chunks:
                if t >= 1:
                    stores[t - 1].wait()
                gathers[t + 1] = start_gather(t + 1)
            gathers[t].wait()
            stores[t] = pltpu.async_copy(
                rb[t % 2],
                out_hbm.at[pl.ds(base + t * GATHER_CHUNK, GATHER_CHUNK)],
                ssem[t % 2])
        stores[n_chunks - 2].wait()
        stores[n_chunks - 1].wait()

    return gather(emb_w, idx)


def kernel(z, emb_w):
    emb_dim = emb_w.shape[1]
    z_flat = z.reshape(-1, emb_dim)
    emb_scaled_t = (emb_w * (-2.0)).T
    idx, loss = _tc_argmin_loss(z_flat, emb_scaled_t)
    z_q = _sc_gather(emb_w, idx)
    return z_q.reshape(z.shape), loss[0], idx.reshape(z.shape[:-1])


# speedup vs baseline: 1.1724x; 1.0003x over previous
import functools

import jax
import jax.numpy as jnp
from jax import lax
from jax.experimental import pallas as pl
from jax.experimental.pallas import tpu as pltpu
from jax.experimental.pallas import tpu_sc as plsc

M_TILE = 4096
N_CHUNK = 4096
LANES = 128

SC_CORES = 2
SC_SUBCORES = 16
SC_WORKERS = SC_CORES * SC_SUBCORES
GATHER_CHUNK = 128


def _tc_body(n_emb, n_rows, z_ref, emb_ref, idx_ref, loss_ref, esq_ref, acc_ref):
    i = pl.program_id(0)
    n_chunks = n_emb // N_CHUNK
    numel = None

    @pl.when(i == 0)
    def _init():
        esq_ref[...] = 0.25 * jnp.sum(emb_ref[...] * emb_ref[...], axis=0)
        acc_ref[0] = 0.0

    z = z_ref[...]
    zsq = jnp.sum(z * z, axis=1)

    v_min = jnp.full((M_TILE, LANES), jnp.inf, dtype=jnp.float32)
    v_col = jnp.zeros((M_TILE, LANES), dtype=jnp.int32)
    for c in range(n_chunks):
        e = emb_ref[:, pl.ds(c * N_CHUNK, N_CHUNK)]
        s2 = lax.dot_general(z, e, (((1,), (0,)), ((), ())),
                             preferred_element_type=jnp.float32)
        esq_c = esq_ref[pl.ds(c * N_CHUNK, N_CHUNK)]
        d = (zsq[:, None] + s2) + esq_c[None, :]
        for k in range(N_CHUNK // LANES):
            dk = d[:, k * LANES:(k + 1) * LANES]
            col = c * (N_CHUNK // LANES) + k
            better = dk < v_min
            v_min = jnp.where(better, dk, v_min)
            v_col = jnp.where(better, col, v_col)

    lane = lax.broadcasted_iota(jnp.int32, (M_TILE, LANES), 1)
    j_full = v_col * LANES + lane
    m = jnp.min(v_min, axis=1)
    idx_row = jnp.min(jnp.where(v_min == m[:, None], j_full, jnp.int32(2 ** 30)),
                      axis=1)
    idx_ref[...] = idx_row
    acc_ref[0] += jnp.sum(m)

    numel = n_rows * z.shape[1]

    @pl.when(i == pl.num_programs(0) - 1)
    def _fin():
        loss_ref[0] = acc_ref[0] * (1.25 / numel)


def _tc_argmin_loss(z_flat, emb_scaled_t):
    n_rows, emb_dim = z_flat.shape
    n_emb = emb_scaled_t.shape[1]
    grid = (n_rows // M_TILE,)
    body = functools.partial(_tc_body, n_emb, n_rows)
    idx, loss = pl.pallas_call(
        body,
        grid=grid,
        in_specs=[
            pl.BlockSpec((M_TILE, emb_dim), lambda i: (i, 0)),
            pl.BlockSpec((emb_dim, n_emb), lambda i: (0, 0)),
        ],
        out_specs=[
            pl.BlockSpec((M_TILE,), lambda i: (i,)),
            pl.BlockSpec(memory_space=pltpu.SMEM),
        ],
        out_shape=[
            jax.ShapeDtypeStruct((n_rows,), jnp.int32),
            jax.ShapeDtypeStruct((1,), jnp.float32),
        ],
        scratch_shapes=[
            pltpu.VMEM((n_emb,), jnp.float32),
            pltpu.SMEM((1,), jnp.float32),
        ],
        compiler_params=pltpu.CompilerParams(
            dimension_semantics=("arbitrary",)),
    )(z_flat, emb_scaled_t)
    return idx, loss


def _sc_gather(emb_w, idx):
    n_rows = idx.shape[0]
    emb_dim = emb_w.shape[1]
    b_per_w = n_rows // SC_WORKERS
    n_chunks = b_per_w // GATHER_CHUNK
    mesh = plsc.VectorSubcoreMesh(core_axis_name="c", subcore_axis_name="s")

    @functools.partial(
        pl.kernel,
        mesh=mesh,
        out_type=jax.ShapeDtypeStruct((n_rows, emb_dim), jnp.float32),
        scratch_types=[
            pltpu.VMEM((b_per_w,), jnp.int32),
            pltpu.VMEM((GATHER_CHUNK, emb_dim), jnp.float32),
            pltpu.VMEM((GATHER_CHUNK, emb_dim), jnp.float32),
            pltpu.SemaphoreType.DMA,
            pltpu.SemaphoreType.DMA,
            pltpu.SemaphoreType.DMA,
            pltpu.SemaphoreType.DMA,
        ],
    )
    def gather(table_hbm, idx_hbm, out_hbm, idx_v, rb0, rb1, g0, g1, s0, s1):
        rb = (rb0, rb1)
        gsem = (g0, g1)
        ssem = (s0, s1)
        wid = lax.axis_index("s") * SC_CORES + lax.axis_index("c")
        base = wid * b_per_w
        pltpu.sync_copy(idx_hbm.at[pl.ds(base, b_per_w)], idx_v)

        def start_gather(t):
            return pltpu.async_copy(
                table_hbm.at[idx_v.at[pl.ds(t * GATHER_CHUNK, GATHER_CHUNK)]],
                rb[t % 2], gsem[t % 2])

        gathers = {0: start_gather(0)}
        stores = {}
        for t in range(n_chunks):
            if t + 1 < n_chunks:
                if t >= 1:
                    stores[t - 1].wait()
                gathers[t + 1] = start_gather(t + 1)
            gathers[t].wait()
            stores[t] = pltpu.async_copy(
                rb[t % 2],
                out_hbm.at[pl.ds(base + t * GATHER_CHUNK, GATHER_CHUNK)],
                ssem[t % 2])
        stores[n_chunks - 2].wait()
        stores[n_chunks - 1].wait()

    return gather(emb_w, idx)


def kernel(z, emb_w):
    emb_dim = emb_w.shape[1]
    z_flat = z.reshape(-1, emb_dim)
    emb_scaled_t = (emb_w * (-2.0)).T
    idx, loss = _tc_argmin_loss(z_flat, emb_scaled_t)
    z_q = _sc_gather(emb_w, idx)
    return z_q.reshape(z.shape), loss[0], idx.reshape(z.shape[:-1])
